# baseline (speedup 1.0000x reference)
import functools

import jax
import jax.numpy as jnp
from jax import lax
from jax.experimental import pallas as pl
from jax.experimental.pallas import tpu as pltpu
from jax.experimental.pallas import tpu_sc as plsc

NTOK = 2048
H = 2048
DFF = 5632
NSLOT = 4

BMT = 1024
BNM = NTOK // BMT
MT = 512
FT = 512
NF = DFF // FT
NM = NTOK // MT

T = 256
NT = NTOK // T + NSLOT
BP = NT * T

_NC = 2
_NW = 32
_BW = BP // _NW
_CH = 16
_NCH = _BW // _CH


def _silu(g):
    return g * jax.nn.sigmoid(g)


def _dot_nt(a, b):
    return jax.lax.dot_general(
        a.astype(jnp.bfloat16), b.astype(jnp.bfloat16),
        (((1,), (1,)), ((), ())), preferred_element_type=jnp.float32)


HP = H // 2
PMT = 512


def _pack_body(x_ref, out_ref):
    xb = x_ref[...]
    lo = lax.bitcast_convert_type(
        xb[:, :HP].astype(jnp.bfloat16), jnp.uint16)
    hi = lax.bitcast_convert_type(
        xb[:, HP:].astype(jnp.bfloat16), jnp.uint16)
    lo32 = lax.convert_element_type(lo, jnp.uint32)
    hi32 = lax.convert_element_type(hi, jnp.uint32)
    out_ref[...] = lax.bitcast_convert_type(
        (hi32 << 16) | lo32, jnp.int32)


def _pack(x):
    return pl.pallas_call(
        _pack_body,
        grid=(NTOK // PMT,),
        in_specs=[pl.BlockSpec((PMT, H), lambda m: (m, 0))],
        out_specs=pl.BlockSpec((PMT, HP), lambda m: (m, 0)),
        out_shape=jax.ShapeDtypeStruct((NTOK, HP), jnp.int32),
        compiler_params=pltpu.CompilerParams(
            dimension_semantics=("arbitrary",)),
    )(x)


def _unpack_bf(u):
    u32 = lax.bitcast_convert_type(u, jnp.uint32)
    lo = lax.bitcast_convert_type(
        lax.convert_element_type(u32 & 0xFFFF, jnp.uint16), jnp.bfloat16)
    hi = lax.bitcast_convert_type(
        lax.convert_element_type(u32 >> 16, jnp.uint16), jnp.bfloat16)
    return lo, hi


def _dot_pk(lo, hi, w):
    return (jax.lax.dot_general(
        lo, w[:, :HP].astype(jnp.bfloat16),
        (((1,), (1,)), ((), ())), preferred_element_type=jnp.float32)
        + jax.lax.dot_general(
        hi, w[:, HP:].astype(jnp.bfloat16),
        (((1,), (1,)), ((), ())), preferred_element_type=jnp.float32))


def _sc_gather_body(x_hbm, gidx_hbm, xg_hbm, idx_v, buf0, buf1, sem0, sem1):
    wid = lax.axis_index("s") * _NC + lax.axis_index("c")
    off = wid * _BW
    pltpu.sync_copy(gidx_hbm.at[pl.ds(off, _BW)], idx_v)
    bufs = (buf0, buf1)
    sems = (sem0, sem1)
    cps = [None, None]
    cps[0] = pltpu.async_copy(x_hbm.at[idx_v.at[pl.ds(0, _CH)]], buf0, sem0)
    for c in range(_NCH):
        if c + 1 < _NCH:
            iv = idx_v.at[pl.ds((c + 1) * _CH, _CH)]
            cps[(c + 1) % 2] = pltpu.async_copy(
                x_hbm.at[iv], bufs[(c + 1) % 2], sems[(c + 1) % 2])
        cps[c % 2].wait()
        pltpu.sync_copy(bufs[c % 2], xg_hbm.at[pl.ds(off + c * _CH, _CH)])


def _gather_rows(x, gidx):
    mesh = plsc.VectorSubcoreMesh(core_axis_name="c", subcore_axis_name="s")
    fn = functools.partial(
        pl.kernel,
        mesh=mesh,
        out_type=jax.ShapeDtypeStruct((BP, HP), jnp.int32),
        scratch_types=[
            pltpu.VMEM((_BW,), jnp.int32),
            pltpu.VMEM((_CH, HP), jnp.int32),
            pltpu.VMEM((_CH, HP), jnp.int32),
            pltpu.SemaphoreType.DMA,
            pltpu.SemaphoreType.DMA,
        ],
    )(_sc_gather_body)
    return fn(x, gidx)


def _adapter_body(sr_ref, xg_ref, rg_ref, ru_ref, rd_ref,
                  fg_ref, fu_ref, fd_ref, scales_ref, out_ref):
    t = pl.program_id(0)
    slot = sr_ref[t]
    rs = scales_ref[slot, 0]
    fs = scales_ref[slot, 1]
    xlo, xhi = _unpack_bf(xg_ref[...])

    hr = _silu(_dot_pk(xlo, xhi, rg_ref[0])) * _dot_pk(xlo, xhi, ru_ref[0]) * rs
    contrib = jax.lax.dot_general(
        hr.astype(jnp.bfloat16), rd_ref[0].astype(jnp.bfloat16),
        (((1,), (1,)), ((), ())), preferred_element_type=jnp.float32)
    hf = _silu(_dot_pk(xlo, xhi, fg_ref[0])) * _dot_pk(xlo, xhi, fu_ref[0]) * fs
    contrib += jax.lax.dot_general(
        hf.astype(jnp.bfloat16), fd_ref[0].astype(jnp.bfloat16),
        (((1,), (1,)), ((), ())), preferred_element_type=jnp.float32)

    out_ref[...] = contrib.astype(jnp.bfloat16)


def _adapter_tiles(tile_slot, xg, retain_gate, retain_up, retain_down,
                   forget_gate, forget_up, forget_down, scales):
    grid_spec = pltpu.PrefetchScalarGridSpec(
        num_scalar_prefetch=1,
        grid=(NT,),
        in_specs=[
            pl.BlockSpec((T, HP), lambda t, sr: (t, 0)),
            pl.BlockSpec((1, 512, H), lambda t, sr: (sr[t], 0, 0)),
            pl.BlockSpec((1, 512, H), lambda t, sr: (sr[t], 0, 0)),
            pl.BlockSpec((1, H, 512), lambda t, sr: (sr[t], 0, 0)),
            pl.BlockSpec((1, 512, H), lambda t, sr: (sr[t], 0, 0)),
            pl.BlockSpec((1, 512, H), lambda t, sr: (sr[t], 0, 0)),
            pl.BlockSpec((1, H, 512), lambda t, sr: (sr[t], 0, 0)),
            pl.BlockSpec(memory_space=pltpu.SMEM),
        ],
        out_specs=pl.BlockSpec((T, H), lambda t, sr: (t, 0)),
    )
    return pl.pallas_call(
        _adapter_body,
        grid_spec=grid_spec,
        out_shape=jax.ShapeDtypeStruct((BP, H), jnp.bfloat16),
        compiler_params=pltpu.CompilerParams(
            dimension_semantics=("arbitrary",)),
    )(tile_slot, xg, retain_gate, retain_up, retain_down,
      forget_gate, forget_up, forget_down, scales)


def _combine_body(q_ref, base_ref, ap_ref, out_ref):
    cols = jax.lax.broadcasted_iota(jnp.int32, (MT, BP), 1)
    onehot = (q_ref[...] == cols).astype(jnp.bfloat16)
    scat = jax.lax.dot_general(
        onehot, ap_ref[...],
        (((1,), (0,)), ((), ())), preferred_element_type=jnp.float32)
    out_ref[...] = base_ref[...] + scat


def _combine(q2, base_out, ap):
    return pl.pallas_call(
        _combine_body,
        grid=(NM,),
        in_specs=[
            pl.BlockSpec((MT, 1), lambda m: (m, 0)),
            pl.BlockSpec((MT, H), lambda m: (m, 0)),
            pl.BlockSpec((BP, H), lambda m: (0, 0)),
        ],
        out_specs=pl.BlockSpec((MT, H), lambda m: (m, 0)),
        out_shape=jax.ShapeDtypeStruct((NTOK, H), jnp.float32),
        compiler_params=pltpu.CompilerParams(
            dimension_semantics=("arbitrary",)),
    )(q2, base_out, ap)


def _base_body(x_ref, gw_ref, uw_ref, dw_ref, out_ref):
    f = pl.program_id(0)
    m = pl.program_id(1)
    xm = x_ref[pl.ds(m * BMT, BMT), :]
    h = _silu(_dot_nt(xm, gw_ref[...])) * _dot_nt(xm, uw_ref[...])
    contrib = jax.lax.dot_general(
        h.astype(jnp.bfloat16), dw_ref[...].astype(jnp.bfloat16),
        (((1,), (1,)), ((), ())), preferred_element_type=jnp.float32)

    @pl.when(f == 0)
    def _():
        out_ref[pl.ds(m * BMT, BMT), :] = contrib

    @pl.when(f != 0)
    def _():
        out_ref[pl.ds(m * BMT, BMT), :] += contrib


def _base_mlp(x_bf, gate_w, up_w, down_w):
    full = pl.BlockSpec((NTOK, H), lambda *_: (0, 0))
    return pl.pallas_call(
        _base_body,
        grid=(NF, BNM),
        in_specs=[
            full,
            pl.BlockSpec((FT, H), lambda f, m: (f, 0)),
            pl.BlockSpec((FT, H), lambda f, m: (f, 0)),
            pl.BlockSpec((H, FT), lambda f, m: (0, f)),
        ],
        out_specs=full,
        out_shape=jax.ShapeDtypeStruct((NTOK, H), jnp.float32),
        compiler_params=pltpu.CompilerParams(
            dimension_semantics=("arbitrary", "arbitrary")),
    )(x_bf, gate_w, up_w, down_w)


def _route_body(ti_ref, q_ref, tslot_ref):
    ti_col = ti_ref[...]
    lane = jax.lax.broadcasted_iota(jnp.int32, (NTOK, 128), 1)
    oh = (ti_col == lane).astype(jnp.bfloat16)
    ri = jax.lax.broadcasted_iota(jnp.int32, (NTOK, NTOK), 0)
    cj = jax.lax.broadcasted_iota(jnp.int32, (NTOK, NTOK), 1)
    tril = (ri > cj).astype(jnp.bfloat16)
    rank2d = jax.lax.dot_general(
        tril, oh, (((1,), (0,)), ((), ())),
        preferred_element_type=jnp.float32)
    rank = jnp.sum(rank2d * oh.astype(jnp.float32), axis=1,
                   keepdims=True)
    counts = rank2d[NTOK - 1:NTOK, :] + oh[NTOK - 1:NTOK, :].astype(
        jnp.float32)
    tiles_per = jnp.floor((counts + (T - 1)) * (1.0 / T))
    li = jax.lax.broadcasted_iota(jnp.int32, (128, 128), 0)
    lj = jax.lax.broadcasted_iota(jnp.int32, (128, 128), 1)
    excl = (li < lj).astype(jnp.float32)
    tile_start = jax.lax.dot_general(
        tiles_per, excl, (((1,), (0,)), ((), ())),
        preferred_element_type=jnp.float32)
    bound = tile_start + tiles_per
    tstart_tok = jnp.sum(tile_start * oh.astype(jnp.float32), axis=1,
                         keepdims=True)
    q_ref[...] = (T * tstart_tok + rank).astype(jnp.int32)
    lane_ok = (lj < NSLOT).astype(jnp.float32)
    ge = (li.astype(jnp.float32) >= jnp.broadcast_to(bound, (128, 128)))
    ts_t = jnp.sum(ge.astype(jnp.float32) * lane_ok, axis=1, keepdims=True)
    tslot_ref[...] = jnp.minimum(ts_t, NSLOT - 1).astype(jnp.int32)


def _route(ti_col):
    return pl.pallas_call(
        _route_body,
        grid=(1,),
        in_specs=[pl.BlockSpec((NTOK, 1), lambda i: (0, 0))],
        out_specs=(pl.BlockSpec((NTOK, 1), lambda i: (0, 0)),
                   pl.BlockSpec((128, 1), lambda i: (0, 0))),
        out_shape=(jax.ShapeDtypeStruct((NTOK, 1), jnp.int32),
                   jax.ShapeDtypeStruct((128, 1), jnp.int32)),
    )(ti_col)


def _routing(ti):
    q2, tslot128 = _route(ti.reshape(NTOK, 1))
    q = q2[:, 0]
    tile_slot = tslot128[:NT, 0]
    row_ids = jnp.full((BP,), NTOK, jnp.int32).at[q].set(
        jnp.arange(NTOK, dtype=jnp.int32))
    gidx = jnp.minimum(row_ids, NTOK - 1)
    return tile_slot, gidx, q2


def kernel(x, token_indices, gate_w, up_w, down_w, retain_gate, retain_up,
           retain_down, forget_gate, forget_up, forget_down, scales):
    ti = token_indices.astype(jnp.int32)
    tile_slot, gidx, q2 = _routing(ti)

    x_bf = x.astype(jnp.bfloat16)
    x_pk = _pack(x)
    xg = _gather_rows(x_pk, gidx)
    base_out = _base_mlp(x_bf, gate_w, up_w, down_w)
    ap = _adapter_tiles(tile_slot, xg,
                        retain_gate, retain_up, retain_down,
                        forget_gate, forget_up, forget_down, scales)
    return _combine(q2, base_out, ap)


# speedup vs baseline: 1.0433x; 1.0433x over previous
import functools

import jax
import jax.numpy as jnp
from jax import lax
from jax.experimental import pallas as pl
from jax.experimental.pallas import tpu as pltpu
from jax.experimental.pallas import tpu_sc as plsc

NTOK = 2048
H = 2048
DFF = 5632
NSLOT = 4

BMT = 1024
BNM = NTOK // BMT
MT = 512
FT = 512
NF = DFF // FT
NM = NTOK // MT

T = 256
NT = NTOK // T + NSLOT
BP = NT * T

_NC = 2
_NW = 32
_BW = BP // _NW
_CH = 16
_NCH = _BW // _CH


def _silu(g):
    return g * jax.nn.sigmoid(g)


def _dot_nt(a, b):
    return jax.lax.dot_general(
        a.astype(jnp.bfloat16), b.astype(jnp.bfloat16),
        (((1,), (1,)), ((), ())), preferred_element_type=jnp.float32)


def _sc_gather_body(x_hbm, gidx_hbm, xg_hbm, idx_v, buf0, buf1, sem0, sem1):
    wid = lax.axis_index("s") * _NC + lax.axis_index("c")
    off = wid * _BW
    pltpu.sync_copy(gidx_hbm.at[pl.ds(off, _BW)], idx_v)
    bufs = (buf0, buf1)
    sems = (sem0, sem1)
    cps = [None, None]
    cps[0] = pltpu.async_copy(x_hbm.at[idx_v.at[pl.ds(0, _CH)]], buf0, sem0)
    for c in range(_NCH):
        if c + 1 < _NCH:
            iv = idx_v.at[pl.ds((c + 1) * _CH, _CH)]
            cps[(c + 1) % 2] = pltpu.async_copy(
                x_hbm.at[iv], bufs[(c + 1) % 2], sems[(c + 1) % 2])
        cps[c % 2].wait()
        pltpu.sync_copy(bufs[c % 2], xg_hbm.at[pl.ds(off + c * _CH, _CH)])


def _gather_rows(x, gidx):
    mesh = plsc.VectorSubcoreMesh(core_axis_name="c", subcore_axis_name="s")
    fn = functools.partial(
        pl.kernel,
        mesh=mesh,
        out_type=jax.ShapeDtypeStruct((BP, H), jnp.float32),
        scratch_types=[
            pltpu.VMEM((_BW,), jnp.int32),
            pltpu.VMEM((_CH, H), jnp.float32),
            pltpu.VMEM((_CH, H), jnp.float32),
            pltpu.SemaphoreType.DMA,
            pltpu.SemaphoreType.DMA,
        ],
    )(_sc_gather_body)
    return fn(x, gidx)


def _adapter_body(sr_ref, xg_ref, rg_ref, ru_ref, rd_ref,
                  fg_ref, fu_ref, fd_ref, scales_ref, out_ref):
    t = pl.program_id(0)
    slot = sr_ref[t]
    rs = scales_ref[slot, 0]
    fs = scales_ref[slot, 1]
    xm = xg_ref[...].astype(jnp.bfloat16)

    hr = _silu(_dot_nt(xm, rg_ref[0])) * _dot_nt(xm, ru_ref[0]) * rs
    contrib = jax.lax.dot_general(
        hr.astype(jnp.bfloat16), rd_ref[0].astype(jnp.bfloat16),
        (((1,), (1,)), ((), ())), preferred_element_type=jnp.float32)
    hf = _silu(_dot_nt(xm, fg_ref[0])) * _dot_nt(xm, fu_ref[0]) * fs
    contrib += jax.lax.dot_general(
        hf.astype(jnp.bfloat16), fd_ref[0].astype(jnp.bfloat16),
        (((1,), (1,)), ((), ())), preferred_element_type=jnp.float32)

    out_ref[...] = contrib.astype(jnp.bfloat16)


def _adapter_tiles(tile_slot, xg, retain_gate, retain_up, retain_down,
                   forget_gate, forget_up, forget_down, scales):
    grid_spec = pltpu.PrefetchScalarGridSpec(
        num_scalar_prefetch=1,
        grid=(NT,),
        in_specs=[
            pl.BlockSpec((T, H), lambda t, sr: (t, 0)),
            pl.BlockSpec((1, 512, H), lambda t, sr: (sr[t], 0, 0)),
            pl.BlockSpec((1, 512, H), lambda t, sr: (sr[t], 0, 0)),
            pl.BlockSpec((1, H, 512), lambda t, sr: (sr[t], 0, 0)),
            pl.BlockSpec((1, 512, H), lambda t, sr: (sr[t], 0, 0)),
            pl.BlockSpec((1, 512, H), lambda t, sr: (sr[t], 0, 0)),
            pl.BlockSpec((1, H, 512), lambda t, sr: (sr[t], 0, 0)),
            pl.BlockSpec(memory_space=pltpu.SMEM),
        ],
        out_specs=pl.BlockSpec((T, H), lambda t, sr: (t, 0)),
    )
    return pl.pallas_call(
        _adapter_body,
        grid_spec=grid_spec,
        out_shape=jax.ShapeDtypeStruct((BP, H), jnp.bfloat16),
        compiler_params=pltpu.CompilerParams(
            dimension_semantics=("arbitrary",)),
    )(tile_slot, xg, retain_gate, retain_up, retain_down,
      forget_gate, forget_up, forget_down, scales)


def _combine_body(q_ref, base_ref, ap_ref, out_ref):
    cols = jax.lax.broadcasted_iota(jnp.int32, (MT, BP), 1)
    onehot = (q_ref[...] == cols).astype(jnp.bfloat16)
    scat = jax.lax.dot_general(
        onehot, ap_ref[...],
        (((1,), (0,)), ((), ())), preferred_element_type=jnp.float32)
    out_ref[...] = base_ref[...] + scat


def _combine(q2, base_out, ap):
    return pl.pallas_call(
        _combine_body,
        grid=(NM,),
        in_specs=[
            pl.BlockSpec((MT, 1), lambda m: (m, 0)),
            pl.BlockSpec((MT, H), lambda m: (m, 0)),
            pl.BlockSpec((BP, H), lambda m: (0, 0)),
        ],
        out_specs=pl.BlockSpec((MT, H), lambda m: (m, 0)),
        out_shape=jax.ShapeDtypeStruct((NTOK, H), jnp.float32),
        compiler_params=pltpu.CompilerParams(
            dimension_semantics=("arbitrary",)),
    )(q2, base_out, ap)


def _base_body(x_ref, gw_ref, uw_ref, dw_ref, out_ref):
    f = pl.program_id(0)
    m = pl.program_id(1)
    xm = x_ref[pl.ds(m * BMT, BMT), :]
    h = _silu(_dot_nt(xm, gw_ref[...])) * _dot_nt(xm, uw_ref[...])
    contrib = jax.lax.dot_general(
        h.astype(jnp.bfloat16), dw_ref[...].astype(jnp.bfloat16),
        (((1,), (1,)), ((), ())), preferred_element_type=jnp.float32)

    @pl.when(f == 0)
    def _():
        out_ref[pl.ds(m * BMT, BMT), :] = contrib

    @pl.when(f != 0)
    def _():
        out_ref[pl.ds(m * BMT, BMT), :] += contrib


def _base_mlp(x_bf, gate_w, up_w, down_w):
    full = pl.BlockSpec((NTOK, H), lambda *_: (0, 0))
    return pl.pallas_call(
        _base_body,
        grid=(NF, BNM),
        in_specs=[
            full,
            pl.BlockSpec((FT, H), lambda f, m: (f, 0)),
            pl.BlockSpec((FT, H), lambda f, m: (f, 0)),
            pl.BlockSpec((H, FT), lambda f, m: (0, f)),
        ],
        out_specs=full,
        out_shape=jax.ShapeDtypeStruct((NTOK, H), jnp.float32),
        compiler_params=pltpu.CompilerParams(
            dimension_semantics=("arbitrary", "arbitrary")),
    )(x_bf, gate_w, up_w, down_w)


def _route_body(ti_ref, q_ref, tslot_ref):
    ti_col = ti_ref[...]
    lane = jax.lax.broadcasted_iota(jnp.int32, (NTOK, 128), 1)
    oh = (ti_col == lane).astype(jnp.bfloat16)
    ri = jax.lax.broadcasted_iota(jnp.int32, (NTOK, NTOK), 0)
    cj = jax.lax.broadcasted_iota(jnp.int32, (NTOK, NTOK), 1)
    tril = (ri > cj).astype(jnp.bfloat16)
    rank2d = jax.lax.dot_general(
        tril, oh, (((1,), (0,)), ((), ())),
        preferred_element_type=jnp.float32)
    rank = jnp.sum(rank2d * oh.astype(jnp.float32), axis=1,
                   keepdims=True)
    counts = rank2d[NTOK - 1:NTOK, :] + oh[NTOK - 1:NTOK, :].astype(
        jnp.float32)
    tiles_per = jnp.floor((counts + (T - 1)) * (1.0 / T))
    li = jax.lax.broadcasted_iota(jnp.int32, (128, 128), 0)
    lj = jax.lax.broadcasted_iota(jnp.int32, (128, 128), 1)
    excl = (li < lj).astype(jnp.float32)
    tile_start = jax.lax.dot_general(
        tiles_per, excl, (((1,), (0,)), ((), ())),
        preferred_element_type=jnp.float32)
    bound = tile_start + tiles_per
    tstart_tok = jnp.sum(tile_start * oh.astype(jnp.float32), axis=1,
                         keepdims=True)
    q_ref[...] = (T * tstart_tok + rank).astype(jnp.int32)
    lane_ok = (lj < NSLOT).astype(jnp.float32)
    ge = (li.astype(jnp.float32) >= jnp.broadcast_to(bound, (128, 128)))
    ts_t = jnp.sum(ge.astype(jnp.float32) * lane_ok, axis=1, keepdims=True)
    tslot_ref[...] = jnp.minimum(ts_t, NSLOT - 1).astype(jnp.int32)


def _route(ti_col):
    return pl.pallas_call(
        _route_body,
        grid=(1,),
        in_specs=[pl.BlockSpec((NTOK, 1), lambda i: (0, 0))],
        out_specs=(pl.BlockSpec((NTOK, 1), lambda i: (0, 0)),
                   pl.BlockSpec((128, 1), lambda i: (0, 0))),
        out_shape=(jax.ShapeDtypeStruct((NTOK, 1), jnp.int32),
                   jax.ShapeDtypeStruct((128, 1), jnp.int32)),
    )(ti_col)


def _routing(ti):
    q2, tslot128 = _route(ti.reshape(NTOK, 1))
    q = q2[:, 0]
    tile_slot = tslot128[:NT, 0]
    row_ids = jnp.full((BP,), NTOK, jnp.int32).at[q].set(
        jnp.arange(NTOK, dtype=jnp.int32))
    gidx = jnp.minimum(row_ids, NTOK - 1)
    return tile_slot, gidx, q2


def kernel(x, token_indices, gate_w, up_w, down_w, retain_gate, retain_up,
           retain_down, forget_gate, forget_up, forget_down, scales):
    ti = token_indices.astype(jnp.int32)
    tile_slot, gidx, q2 = _routing(ti)

    x_bf = x.astype(jnp.bfloat16)
    xg = _gather_rows(x, gidx)
    base_out = _base_mlp(x_bf, gate_w, up_w, down_w)
    ap = _adapter_tiles(tile_slot, xg,
                        retain_gate, retain_up, retain_down,
                        forget_gate, forget_up, forget_down, scales)
    return _combine(q2, base_out, ap)
